# baseline (speedup 1.0000x reference)
import jax
import jax.numpy as jnp
from jax import lax
from jax.experimental import pallas as pl
from jax.experimental.pallas import tpu as pltpu
from jax.experimental.pallas import tpu_sc as plsc

N = 10000
D = 128
E = 320000
H = 128
DH = 64
NS = 16
NC = 2
CW = 128
NCHUNK = 160
EPT = NCHUNK * CW
EPAD = NS * EPT
NPAD = 10240
RPT = NPAD // NS
TB = 1000


def _sc_body(x2, rowh, colh, valh, outax, outax2, outdeg,
             rowv, colv, valv, gba, ob1, zbuf,
             acc, accd, gsa):
    c = lax.axis_index("c")
    s = lax.axis_index("s")
    base = s * RPT

    pltpu.sync_copy(rowh.at[s], rowv)
    pltpu.sync_copy(colh.at[c, s], colv)
    pltpu.sync_copy(valh.at[s], valv)

    zero16 = jnp.zeros((16,), jnp.float32)

    def zdeg(i, _):
        zbuf[pl.ds(i * 16, 16)] = zero16
        return 0

    lax.fori_loop(0, RPT // 16, zdeg, 0)

    def do_phase(second):
        def zrow(i, _):
            ob1[i // 4, pl.ds((i % 4) * 16, 16)] = zero16
            return 0

        lax.fori_loop(0, CW * 4, zrow, 0)
        for k in range(RPT // CW):
            pltpu.sync_copy(ob1, acc.at[pl.ds(base + k * CW, CW)])
        if not second:
            pltpu.sync_copy(zbuf, accd.at[pl.ds(base, RPT)])
        plsc.subcore_barrier()

        def compute(j):
            def group(gb, _):
                eb = gb * 16
                val16 = valv[j, pl.ds(eb, 16)]
                for k in range(16):
                    e = eb + k
                    v = val16[k]
                    for q in range(4):
                        gv = gba[e, pl.ds(q * 16, 16)]
                        a = gv * v
                        if second:
                            a = a * gv
                        ob1[e, pl.ds(q * 16, 16)] = a
                return 0

            lax.fori_loop(0, CW // 16, group, 0)

        def scatters(j):
            pltpu.sync_copy(ob1, acc.at[rowv.at[j]], add=True)
            if not second:
                pltpu.sync_copy(valv.at[j], accd.at[rowv.at[j]], add=True)

        def chunk(j, _):
            pltpu.async_copy(x2.at[colv.at[j]], gba, gsa).wait()
            compute(j)
            scatters(j)
            return 0

        lax.fori_loop(0, NCHUNK, chunk, 0)
        plsc.subcore_barrier()

        outref = outax2 if second else outax
        for k in range(RPT // CW):
            pltpu.sync_copy(acc.at[pl.ds(base + k * CW, CW)], ob1)
            pltpu.sync_copy(ob1, outref.at[c, pl.ds(base + k * CW, CW)])
        if not second:
            pltpu.sync_copy(accd.at[pl.ds(base, RPT)], zbuf)
            pltpu.sync_copy(zbuf, outdeg.at[c, pl.ds(base, RPT)])

    do_phase(False)
    do_phase(True)


def _tc_body(x_ref, ax_ref, ax2_ref, deg_ref, w1a, w1b, w1c, w1d,
             b1r, gr, br, w2a, w2b, b2r, out_ref):
    x = x_ref[...]
    ax = ax_ref[...]
    ax2 = ax2_ref[...]
    deg = deg_ref[...]
    rdeg = 1.0 / (deg + 1e-8)
    mean = ax * rdeg
    var = ax2 * rdeg - mean * mean
    std = jnp.sqrt(jnp.maximum(var, 0.0))
    delta = ax - x
    h = jnp.dot(x, w1a[...], preferred_element_type=jnp.float32)
    h = h + jnp.dot(delta, w1b[...], preferred_element_type=jnp.float32)
    h = h + jnp.dot(std, w1c[...], preferred_element_type=jnp.float32)
    h = h + deg * w1d[...] + b1r[...]
    mu = jnp.mean(h, axis=1, keepdims=True)
    hc = h - mu
    v = jnp.mean(hc * hc, axis=1, keepdims=True)
    hn = hc * lax.rsqrt(v + 1e-5) * gr[...] + br[...]
    h2 = jnp.maximum(hn, 0.0)
    b2 = b2r[...]
    l0 = jnp.sum(h2 * w2a[...], axis=1, keepdims=True) + b2[0:1, 0:1]
    l1 = jnp.sum(h2 * w2b[...], axis=1, keepdims=True) + b2[0:1, 1:2]
    p0 = 1.0 / (1.0 + jnp.exp(l1 - l0))
    p1 = 1.0 / (1.0 + jnp.exp(l0 - l1))
    out_ref[...] = jnp.concatenate([p0, p1], axis=1)


_seg = pl.kernel(
    _sc_body,
    out_type=(
        jax.ShapeDtypeStruct((NC, NPAD, DH), jnp.float32),
        jax.ShapeDtypeStruct((NC, NPAD, DH), jnp.float32),
        jax.ShapeDtypeStruct((NC, NPAD), jnp.float32),
    ),
    mesh=plsc.VectorSubcoreMesh(
        core_axis_name="c", subcore_axis_name="s",
        num_cores=NC, num_subcores=NS),
    compiler_params=pltpu.CompilerParams(use_tc_tiling_on_sc=False),
    scratch_types=[
        pltpu.VMEM((NCHUNK, CW), jnp.int32),
        pltpu.VMEM((NCHUNK, CW), jnp.int32),
        pltpu.VMEM((NCHUNK, CW), jnp.float32),
        pltpu.VMEM((CW, DH), jnp.float32),
        pltpu.VMEM((CW, DH), jnp.float32),
        pltpu.VMEM((RPT,), jnp.float32),
        pltpu.VMEM_SHARED((NPAD, DH), jnp.float32),
        pltpu.VMEM_SHARED((NPAD,), jnp.float32),
        pltpu.SemaphoreType.DMA,
    ],
)


def kernel(x, adj_vals, edge_index, W1, b1, gamma, beta, W2, b2):
    row = edge_index[0]
    col = edge_index[1]
    pad = EPAD - E
    rowh = jnp.pad(row, (0, pad)).reshape(NS, NCHUNK, CW)
    valh = jnp.pad(adj_vals, (0, pad)).reshape(NS, NCHUNK, CW)
    colp = jnp.pad(col, (0, pad))
    colh = jnp.stack([2 * colp, 2 * colp + 1]).reshape(NC, NS, NCHUNK, CW)
    x2 = x.reshape(2 * N, DH)

    outax, outax2, outdeg = _seg(x2, rowh, colh, valh)

    AX = jnp.concatenate([outax[0, :N], outax[1, :N]], axis=1)
    AX2 = jnp.concatenate([outax2[0, :N], outax2[1, :N]], axis=1)
    deg2 = outdeg[0, :N].reshape(N, 1)

    W1a = W1[0:D]
    W1b = W1[D:2 * D]
    W1c = W1[2 * D:3 * D]
    w1d = W1[3 * D:3 * D + 1]
    b1r = b1.reshape(1, H)
    gr = gamma.reshape(1, H)
    br = beta.reshape(1, H)
    w2a = W2[:, 0].reshape(1, H)
    w2b = W2[:, 1].reshape(1, H)
    b2r = b2.reshape(1, 2)

    wspec = pl.BlockSpec((D, H), lambda i: (0, 0))
    rspec = pl.BlockSpec((1, H), lambda i: (0, 0))
    out = pl.pallas_call(
        _tc_body,
        grid=(N // TB,),
        in_specs=[
            pl.BlockSpec((TB, D), lambda i: (i, 0)),
            pl.BlockSpec((TB, D), lambda i: (i, 0)),
            pl.BlockSpec((TB, D), lambda i: (i, 0)),
            pl.BlockSpec((TB, 1), lambda i: (i, 0)),
            wspec, wspec, wspec, rspec,
            rspec, rspec, rspec,
            rspec, rspec,
            pl.BlockSpec((1, 2), lambda i: (0, 0)),
        ],
        out_specs=pl.BlockSpec((TB, 2), lambda i: (i, 0)),
        out_shape=jax.ShapeDtypeStruct((N, 2), jnp.float32),
    )(x, AX, AX2, deg2, W1a, W1b, W1c, w1d, b1r, gr, br, w2a, w2b, b2r)
    return out


# speedup vs baseline: 1.4743x; 1.4743x over previous
import jax
import jax.numpy as jnp
from jax import lax
from jax.experimental import pallas as pl
from jax.experimental.pallas import tpu as pltpu
from jax.experimental.pallas import tpu_sc as plsc

N = 10000
D = 128
E = 320000
H = 128
DH = 64
NS = 16
NC = 2
CW = 128
NCHUNK = 157
EPT = NCHUNK * CW
EPAD = NS * EPT
NPAD = 10240
RPT = NPAD // NS
TB = 1000


def _sc_body(x2, rowh, colh, valh, outax, outax2, outdeg,
             rowv, colv, valv, gba, ob1, zbuf,
             acc, accd, gsa):
    c = lax.axis_index("c")
    s = lax.axis_index("s")
    base = s * RPT

    pltpu.sync_copy(rowh.at[s], rowv)
    pltpu.sync_copy(colh.at[c, s], colv)
    pltpu.sync_copy(valh.at[s], valv)

    zero16 = jnp.zeros((16,), jnp.float32)

    def zdeg(i, _):
        zbuf[pl.ds(i * 16, 16)] = zero16
        return 0

    lax.fori_loop(0, RPT // 16, zdeg, 0)

    def do_phase(second):
        def zrow(i, _):
            ob1[i // 4, pl.ds((i % 4) * 16, 16)] = zero16
            return 0

        lax.fori_loop(0, CW * 4, zrow, 0)
        for k in range(RPT // CW):
            pltpu.sync_copy(ob1, acc.at[pl.ds(base + k * CW, CW)])
        if not second:
            pltpu.sync_copy(zbuf, accd.at[pl.ds(base, RPT)])
        plsc.subcore_barrier()

        def compute(j):
            def group(gb, _):
                eb = gb * 16
                val16 = valv[j, pl.ds(eb, 16)]
                for k in range(16):
                    e = eb + k
                    v = val16[k]
                    for q in range(4):
                        gv = gba[e, pl.ds(q * 16, 16)]
                        a = gv * v
                        if second:
                            a = a * gv
                        ob1[e, pl.ds(q * 16, 16)] = a
                return 0

            lax.fori_loop(0, CW // 16, group, 0)

        def scatters(j):
            pltpu.sync_copy(ob1, acc.at[rowv.at[j]], add=True)
            if not second:
                pltpu.sync_copy(valv.at[j], accd.at[rowv.at[j]], add=True)

        def chunk(j, _):
            pltpu.async_copy(x2.at[colv.at[j]], gba, gsa).wait()
            compute(j)
            scatters(j)
            return 0

        lax.fori_loop(0, NCHUNK, chunk, 0)
        plsc.subcore_barrier()

        outref = outax2 if second else outax
        for k in range(RPT // CW):
            pltpu.sync_copy(acc.at[pl.ds(base + k * CW, CW)], ob1)
            pltpu.sync_copy(ob1, outref.at[c, pl.ds(base + k * CW, CW)])
        if not second:
            pltpu.sync_copy(accd.at[pl.ds(base, RPT)], zbuf)
            pltpu.sync_copy(zbuf, outdeg.at[c, pl.ds(base, RPT)])

    do_phase(False)
    do_phase(True)


def _tc_body(x_ref, ax_ref, ax2_ref, deg_ref, w1a, w1b, w1c, w1d,
             b1r, gr, br, w2a, w2b, b2r, out_ref):
    x = x_ref[...]
    ax = ax_ref[...]
    ax2 = ax2_ref[...]
    deg = deg_ref[...]
    rdeg = 1.0 / (deg + 1e-8)
    mean = ax * rdeg
    var = ax2 * rdeg - mean * mean
    std = jnp.sqrt(jnp.maximum(var, 0.0))
    delta = ax - x
    h = jnp.dot(x, w1a[...], preferred_element_type=jnp.float32)
    h = h + jnp.dot(delta, w1b[...], preferred_element_type=jnp.float32)
    h = h + jnp.dot(std, w1c[...], preferred_element_type=jnp.float32)
    h = h + deg * w1d[...] + b1r[...]
    mu = jnp.mean(h, axis=1, keepdims=True)
    hc = h - mu
    v = jnp.mean(hc * hc, axis=1, keepdims=True)
    hn = hc * lax.rsqrt(v + 1e-5) * gr[...] + br[...]
    h2 = jnp.maximum(hn, 0.0)
    b2 = b2r[...]
    l0 = jnp.sum(h2 * w2a[...], axis=1, keepdims=True) + b2[0:1, 0:1]
    l1 = jnp.sum(h2 * w2b[...], axis=1, keepdims=True) + b2[0:1, 1:2]
    p0 = 1.0 / (1.0 + jnp.exp(l1 - l0))
    p1 = 1.0 / (1.0 + jnp.exp(l0 - l1))
    out_ref[...] = jnp.concatenate([p0, p1], axis=1)


_seg = pl.kernel(
    _sc_body,
    out_type=(
        jax.ShapeDtypeStruct((NC, NPAD, DH), jnp.float32),
        jax.ShapeDtypeStruct((NC, NPAD, DH), jnp.float32),
        jax.ShapeDtypeStruct((NC, NPAD), jnp.float32),
    ),
    mesh=plsc.VectorSubcoreMesh(
        core_axis_name="c", subcore_axis_name="s",
        num_cores=NC, num_subcores=NS),
    compiler_params=pltpu.CompilerParams(use_tc_tiling_on_sc=False),
    scratch_types=[
        pltpu.VMEM((NCHUNK, CW), jnp.int32),
        pltpu.VMEM((NCHUNK, CW), jnp.int32),
        pltpu.VMEM((NCHUNK, CW), jnp.float32),
        pltpu.VMEM((CW, DH), jnp.float32),
        pltpu.VMEM((CW, DH), jnp.float32),
        pltpu.VMEM((RPT,), jnp.float32),
        pltpu.VMEM_SHARED((NPAD, DH), jnp.float32),
        pltpu.VMEM_SHARED((NPAD,), jnp.float32),
        pltpu.SemaphoreType.DMA,
    ],
)


def kernel(x, adj_vals, edge_index, W1, b1, gamma, beta, W2, b2):
    row = edge_index[0]
    col = edge_index[1]
    pad = EPAD - E
    rowh = jnp.pad(row, (0, pad)).reshape(NS, NCHUNK, CW)
    valh = jnp.pad(adj_vals, (0, pad)).reshape(NS, NCHUNK, CW)
    colp = jnp.pad(col, (0, pad))
    colh = jnp.stack([2 * colp, 2 * colp + 1]).reshape(NC, NS, NCHUNK, CW)
    x2 = x.reshape(2 * N, DH)

    outax, outax2, outdeg = _seg(x2, rowh, colh, valh)

    AX = jnp.concatenate([outax[0, :N], outax[1, :N]], axis=1)
    AX2 = jnp.concatenate([outax2[0, :N], outax2[1, :N]], axis=1)
    deg2 = outdeg[0, :N].reshape(N, 1)

    W1a = W1[0:D]
    W1b = W1[D:2 * D]
    W1c = W1[2 * D:3 * D]
    w1d = W1[3 * D:3 * D + 1]
    b1r = b1.reshape(1, H)
    gr = gamma.reshape(1, H)
    br = beta.reshape(1, H)
    w2a = W2[:, 0].reshape(1, H)
    w2b = W2[:, 1].reshape(1, H)
    b2r = b2.reshape(1, 2)

    wspec = pl.BlockSpec((D, H), lambda i: (0, 0))
    rspec = pl.BlockSpec((1, H), lambda i: (0, 0))
    out = pl.pallas_call(
        _tc_body,
        grid=(N // TB,),
        in_specs=[
            pl.BlockSpec((TB, D), lambda i: (i, 0)),
            pl.BlockSpec((TB, D), lambda i: (i, 0)),
            pl.BlockSpec((TB, D), lambda i: (i, 0)),
            pl.BlockSpec((TB, 1), lambda i: (i, 0)),
            wspec, wspec, wspec, rspec,
            rspec, rspec, rspec,
            rspec, rspec,
            pl.BlockSpec((1, 2), lambda i: (0, 0)),
        ],
        out_specs=pl.BlockSpec((TB, 2), lambda i: (i, 0)),
        out_shape=jax.ShapeDtypeStruct((N, 2), jnp.float32),
    )(x, AX, AX2, deg2, W1a, W1b, W1c, w1d, b1r, gr, br, w2a, w2b, b2r)
    return out


# speedup vs baseline: 1.7501x; 1.1871x over previous
import jax
import jax.numpy as jnp
from jax import lax
from jax.experimental import pallas as pl
from jax.experimental.pallas import tpu as pltpu
from jax.experimental.pallas import tpu_sc as plsc

N = 10000
D = 128
E = 320000
H = 128
DH = 64
NS = 16
NC = 2
CW = 128
NCHUNK = 157
EPT = NCHUNK * CW
EPAD = NS * EPT
NPAD = 10240
RPT = NPAD // NS
TB = 1000


def _sc_body(x2, rowh, colh, valh, outax, outax2, outdeg,
             rowv, colv, valv, gba, ob1, zbuf,
             acc, accd, gsa):
    c = lax.axis_index("c")
    s = lax.axis_index("s")
    base = s * RPT

    pltpu.sync_copy(rowh.at[s], rowv)
    pltpu.sync_copy(colh.at[c, s], colv)
    pltpu.sync_copy(valh.at[s], valv)

    zero16 = jnp.zeros((16,), jnp.float32)

    def zdeg(i, _):
        zbuf[pl.ds(i * 16, 16)] = zero16
        return 0

    lax.fori_loop(0, RPT // 16, zdeg, 0)

    def do_phase(second):
        def zrow(i, _):
            ob1[i // 4, pl.ds((i % 4) * 16, 16)] = zero16
            return 0

        lax.fori_loop(0, CW * 4, zrow, 0)
        for k in range(RPT // CW):
            pltpu.sync_copy(ob1, acc.at[pl.ds(base + k * CW, CW)])
        if not second:
            pltpu.sync_copy(zbuf, accd.at[pl.ds(base, RPT)])
        plsc.subcore_barrier()

        def compute(j):
            def group(gb, _):
                eb = gb * 16
                val16 = valv[j, pl.ds(eb, 16)]
                for k in range(16):
                    e = eb + k
                    v = val16[k]
                    for q in range(4):
                        gv = gba[e, pl.ds(q * 16, 16)]
                        a = gv * v
                        if second:
                            a = a * gv
                        ob1[e, pl.ds(q * 16, 16)] = a
                return 0

            lax.fori_loop(0, CW // 16, group, 0)

        def scatters(j):
            pltpu.sync_copy(ob1, acc.at[rowv.at[j]], add=True)
            if not second:
                pltpu.sync_copy(valv.at[j], accd.at[rowv.at[j]], add=True)

        pltpu.async_copy(x2.at[colv.at[0]], gba, gsa)

        def chunk(j, _):
            pltpu.make_async_copy(x2.at[colv.at[j]], gba, gsa).wait()
            compute(j)
            pltpu.async_copy(x2.at[colv.at[j + 1]], gba, gsa)
            scatters(j)
            return 0

        lax.fori_loop(0, NCHUNK - 1, chunk, 0)
        pltpu.make_async_copy(x2.at[colv.at[NCHUNK - 1]], gba, gsa).wait()
        compute(NCHUNK - 1)
        scatters(NCHUNK - 1)
        plsc.subcore_barrier()

        outref = outax2 if second else outax
        for k in range(RPT // CW):
            pltpu.sync_copy(acc.at[pl.ds(base + k * CW, CW)], ob1)
            pltpu.sync_copy(ob1, outref.at[c, pl.ds(base + k * CW, CW)])
        if not second:
            pltpu.sync_copy(accd.at[pl.ds(base, RPT)], zbuf)
            pltpu.sync_copy(zbuf, outdeg.at[c, pl.ds(base, RPT)])

    do_phase(False)
    do_phase(True)


def _tc_body(x_ref, ax_ref, ax2_ref, deg_ref, w1a, w1b, w1c, w1d,
             b1r, gr, br, w2a, w2b, b2r, out_ref):
    x = x_ref[...]
    ax = ax_ref[...]
    ax2 = ax2_ref[...]
    deg = deg_ref[...]
    rdeg = 1.0 / (deg + 1e-8)
    mean = ax * rdeg
    var = ax2 * rdeg - mean * mean
    std = jnp.sqrt(jnp.maximum(var, 0.0))
    delta = ax - x
    h = jnp.dot(x, w1a[...], preferred_element_type=jnp.float32)
    h = h + jnp.dot(delta, w1b[...], preferred_element_type=jnp.float32)
    h = h + jnp.dot(std, w1c[...], preferred_element_type=jnp.float32)
    h = h + deg * w1d[...] + b1r[...]
    mu = jnp.mean(h, axis=1, keepdims=True)
    hc = h - mu
    v = jnp.mean(hc * hc, axis=1, keepdims=True)
    hn = hc * lax.rsqrt(v + 1e-5) * gr[...] + br[...]
    h2 = jnp.maximum(hn, 0.0)
    b2 = b2r[...]
    l0 = jnp.sum(h2 * w2a[...], axis=1, keepdims=True) + b2[0:1, 0:1]
    l1 = jnp.sum(h2 * w2b[...], axis=1, keepdims=True) + b2[0:1, 1:2]
    p0 = 1.0 / (1.0 + jnp.exp(l1 - l0))
    p1 = 1.0 / (1.0 + jnp.exp(l0 - l1))
    out_ref[...] = jnp.concatenate([p0, p1], axis=1)


_seg = pl.kernel(
    _sc_body,
    out_type=(
        jax.ShapeDtypeStruct((NC, NPAD, DH), jnp.float32),
        jax.ShapeDtypeStruct((NC, NPAD, DH), jnp.float32),
        jax.ShapeDtypeStruct((NC, NPAD), jnp.float32),
    ),
    mesh=plsc.VectorSubcoreMesh(
        core_axis_name="c", subcore_axis_name="s",
        num_cores=NC, num_subcores=NS),
    compiler_params=pltpu.CompilerParams(use_tc_tiling_on_sc=False),
    scratch_types=[
        pltpu.VMEM((NCHUNK, CW), jnp.int32),
        pltpu.VMEM((NCHUNK, CW), jnp.int32),
        pltpu.VMEM((NCHUNK, CW), jnp.float32),
        pltpu.VMEM((CW, DH), jnp.float32),
        pltpu.VMEM((CW, DH), jnp.float32),
        pltpu.VMEM((RPT,), jnp.float32),
        pltpu.VMEM_SHARED((NPAD, DH), jnp.float32),
        pltpu.VMEM_SHARED((NPAD,), jnp.float32),
        pltpu.SemaphoreType.DMA,
    ],
)


def kernel(x, adj_vals, edge_index, W1, b1, gamma, beta, W2, b2):
    row = edge_index[0]
    col = edge_index[1]
    pad = EPAD - E
    rowh = jnp.pad(row, (0, pad)).reshape(NS, NCHUNK, CW)
    valh = jnp.pad(adj_vals, (0, pad)).reshape(NS, NCHUNK, CW)
    colp = jnp.pad(col, (0, pad))
    colh = jnp.stack([2 * colp, 2 * colp + 1]).reshape(NC, NS, NCHUNK, CW)
    x2 = x.reshape(2 * N, DH)

    outax, outax2, outdeg = _seg(x2, rowh, colh, valh)

    AX = jnp.concatenate([outax[0, :N], outax[1, :N]], axis=1)
    AX2 = jnp.concatenate([outax2[0, :N], outax2[1, :N]], axis=1)
    deg2 = outdeg[0, :N].reshape(N, 1)

    W1a = W1[0:D]
    W1b = W1[D:2 * D]
    W1c = W1[2 * D:3 * D]
    w1d = W1[3 * D:3 * D + 1]
    b1r = b1.reshape(1, H)
    gr = gamma.reshape(1, H)
    br = beta.reshape(1, H)
    w2a = W2[:, 0].reshape(1, H)
    w2b = W2[:, 1].reshape(1, H)
    b2r = b2.reshape(1, 2)

    wspec = pl.BlockSpec((D, H), lambda i: (0, 0))
    rspec = pl.BlockSpec((1, H), lambda i: (0, 0))
    out = pl.pallas_call(
        _tc_body,
        grid=(N // TB,),
        in_specs=[
            pl.BlockSpec((TB, D), lambda i: (i, 0)),
            pl.BlockSpec((TB, D), lambda i: (i, 0)),
            pl.BlockSpec((TB, D), lambda i: (i, 0)),
            pl.BlockSpec((TB, 1), lambda i: (i, 0)),
            wspec, wspec, wspec, rspec,
            rspec, rspec, rspec,
            rspec, rspec,
            pl.BlockSpec((1, 2), lambda i: (0, 0)),
        ],
        out_specs=pl.BlockSpec((TB, 2), lambda i: (i, 0)),
        out_shape=jax.ShapeDtypeStruct((N, 2), jnp.float32),
    )(x, AX, AX2, deg2, W1a, W1b, W1c, w1d, b1r, gr, br, w2a, w2b, b2r)
    return out
